# baseline (device time: 32960 ns/iter reference)
import jax
import jax.numpy as jnp
from jax import lax
from jax.experimental import pallas as pl
from jax.experimental.pallas import tpu as pltpu

M = 1024
D = 1024
NCH = 8
CH = M // NCH
S = 4
SUB = CH // S
EPS = 1e-6
_MESH = pl.DeviceIdType.MESH

_X1, _ON, _OP, _OT, _AF, _BF, _TA, _TB = (i * S for i in range(8))


def kernel(partial, resid, gamma):
    gamma2d = gamma.reshape(1, D)

    def body(partial_ref, resid_ref, gamma_ref, out_ref,
             xrecv, stage, send_sems, recv_sems):
        x = lax.axis_index("x")
        y = lax.axis_index("y")
        z = lax.axis_index("z")
        p = jnp.where(y == 0, z, 7 - z)
        c_own = (p + 4 * x) % 8
        c_twin = (c_own + 4) % 8

        def plane_coords(q):
            return (q // 4, jnp.where(q < 4, q, 7 - q))

        twin = (1 - x, y, z)
        ny, nz = plane_coords((p + 1) % 8)
        nxt = (x, ny, nz)
        py, pz = plane_coords((p + 7) % 8)
        prv = (x, py, pz)

        def orows(chunk, j):
            return out_ref.at[pl.ds((chunk % 8) * CH + j * SUB, SUB), :]

        def rdma(src, dst, sem_idx, dev):
            return pltpu.make_async_remote_copy(
                src_ref=src, dst_ref=dst,
                send_sem=send_sems.at[sem_idx],
                recv_sem=recv_sems.at[sem_idx],
                device_id=dev, device_id_type=_MESH,
            )

        barrier = pltpu.get_barrier_semaphore()
        for nbr in (twin, nxt, prv):
            pl.semaphore_signal(barrier, inc=1, device_id=nbr,
                                device_id_type=_MESH)
        pl.semaphore_wait(barrier, 3)

        stage[...] = partial_ref[0, pl.ds(c_twin * CH, CH), :]
        x1 = [rdma(stage.at[pl.ds(j * SUB, SUB), :],
                   xrecv.at[pl.ds(j * SUB, SUB), :], _X1 + j, twin)
              for j in range(S)]
        for d in x1:
            d.start()

        own_n, own_p, own_t = [], [], []
        for j in range(S):
            x1[j].wait_recv()
            rows_oj = pl.ds(c_own * CH + j * SUB, SUB)
            ysum = (partial_ref[0, rows_oj, :]
                    + xrecv[pl.ds(j * SUB, SUB), :]
                    + resid_ref[rows_oj, :])
            ms = jnp.mean(ysum * ysum, axis=-1, keepdims=True)
            out_ref[rows_oj, :] = ysum * lax.rsqrt(ms + EPS) * gamma_ref[...]

            o = orows(c_own, j)
            own_n.append(rdma(o, o, _ON + j, nxt))
            own_p.append(rdma(o, o, _OP + j, prv))
            own_t.append(rdma(o, o, _OT + j, twin))
            own_n[j].start()
            own_p[j].start()
            own_t[j].start()

        afwd, bfwd = [], []
        for j in range(S):
            own_n[j].wait_recv()
            o = orows(c_own + 7, j)
            afwd.append(rdma(o, o, _AF + j, nxt))
            afwd[j].start()
            own_p[j].wait_recv()
            o = orows(c_own + 1, j)
            bfwd.append(rdma(o, o, _BF + j, prv))
            bfwd[j].start()

        twf_n, twf_p = [], []
        for j in range(S):
            own_t[j].wait_recv()
            o = orows(c_own + 4, j)
            twf_n.append(rdma(o, o, _TA + j, nxt))
            twf_p.append(rdma(o, o, _TB + j, prv))
            twf_n[j].start()
            twf_p[j].start()

        for j in range(S):
            afwd[j].wait_recv()
            bfwd[j].wait_recv()
            twf_n[j].wait_recv()
            twf_p[j].wait_recv()

        for d in x1 + own_n + own_p + own_t + afwd + bfwd + twf_n + twf_p:
            d.wait_send()

    return pl.pallas_call(
        body,
        out_shape=jax.ShapeDtypeStruct((M, D), jnp.float32),
        in_specs=[
            pl.BlockSpec(memory_space=pltpu.VMEM),
            pl.BlockSpec(memory_space=pltpu.VMEM),
            pl.BlockSpec(memory_space=pltpu.VMEM),
        ],
        out_specs=pl.BlockSpec(memory_space=pltpu.VMEM),
        scratch_shapes=[
            pltpu.VMEM((CH, D), jnp.float32),
            pltpu.VMEM((CH, D), jnp.float32),
            pltpu.SemaphoreType.DMA((8 * S,)),
            pltpu.SemaphoreType.DMA((8 * S,)),
        ],
        compiler_params=pltpu.CompilerParams(collective_id=0),
    )(partial, resid, gamma2d)


# device time: 31570 ns/iter; 1.0440x vs baseline; 1.0440x over previous
import jax
import jax.numpy as jnp
from jax import lax
from jax.experimental import pallas as pl
from jax.experimental.pallas import tpu as pltpu

M = 1024
D = 1024
NCH = 8
CH = M // NCH
S = 4
SUB = CH // S
EPS = 1e-6
_MESH = pl.DeviceIdType.MESH

_X1, _ON, _OP, _OT, _AF, _BF, _TA, _TB, _R3, _R5 = (i * S for i in range(10))


def kernel(partial, resid, gamma):
    gamma2d = gamma.reshape(1, D)

    def body(partial_ref, resid_ref, gamma_ref, out_ref,
             xrecv, send_sems, recv_sems):
        x = lax.axis_index("x")
        y = lax.axis_index("y")
        z = lax.axis_index("z")
        p = jnp.where(y == 0, z, 7 - z)
        c_own = (p + 4 * x) % 8
        c_twin = (c_own + 4) % 8

        def plane_coords(q):
            return (q // 4, jnp.where(q < 4, q, 7 - q))

        twin = (1 - x, y, z)
        ny, nz = plane_coords((p + 1) % 8)
        nxt = (x, ny, nz)
        py, pz = plane_coords((p + 7) % 8)
        prv = (x, py, pz)

        def orows(chunk, j):
            return out_ref.at[pl.ds((chunk % 8) * CH + j * SUB, SUB), :]

        def rdma(src, dst, sem_idx, dev):
            return pltpu.make_async_remote_copy(
                src_ref=src, dst_ref=dst,
                send_sem=send_sems.at[sem_idx],
                recv_sem=recv_sems.at[sem_idx],
                device_id=dev, device_id_type=_MESH,
            )

        barrier = pltpu.get_barrier_semaphore()
        for nbr in (twin, nxt, prv):
            pl.semaphore_signal(barrier, inc=1, device_id=nbr,
                                device_id_type=_MESH)
        pl.semaphore_wait(barrier, 3)

        x1 = [rdma(partial_ref.at[0, pl.ds(c_twin * CH + j * SUB, SUB), :],
                   xrecv.at[pl.ds(j * SUB, SUB), :], _X1 + j, twin)
              for j in range(S)]
        for d in x1:
            d.start()

        own_n, own_p, own_t = [], [], []
        for j in range(S):
            x1[j].wait_recv()
            rows_oj = pl.ds(c_own * CH + j * SUB, SUB)
            ysum = (partial_ref[0, rows_oj, :]
                    + xrecv[pl.ds(j * SUB, SUB), :]
                    + resid_ref[rows_oj, :])
            ms = jnp.mean(ysum * ysum, axis=-1, keepdims=True)
            out_ref[rows_oj, :] = ysum * lax.rsqrt(ms + EPS) * gamma_ref[...]

            o = orows(c_own, j)
            own_n.append(rdma(o, o, _ON + j, nxt))
            own_p.append(rdma(o, o, _OP + j, prv))
            own_t.append(rdma(o, o, _OT + j, twin))
            own_n[j].start()
            own_p[j].start()
            own_t[j].start()

        afwd, bfwd = [], []
        xr3, xr5 = {}, {}
        for j in range(S):
            own_n[j].wait_recv()
            o = orows(c_own + 7, j)
            afwd.append(rdma(o, o, _AF + j, nxt))
            afwd[j].start()
            if j >= 2:
                xr3[j] = rdma(o, o, _R3 + j, twin)
                xr3[j].start()
            own_p[j].wait_recv()
            o = orows(c_own + 1, j)
            bfwd.append(rdma(o, o, _BF + j, prv))
            bfwd[j].start()
            if j == 3:
                xr5[j] = rdma(o, o, _R5 + j, twin)
                xr5[j].start()

        twf_n, twf_p = {}, {}
        for j in range(S):
            own_t[j].wait_recv()
            o = orows(c_own + 4, j)
            if j < 2:
                twf_n[j] = rdma(o, o, _TA + j, nxt)
                twf_n[j].start()
            if j < 3:
                twf_p[j] = rdma(o, o, _TB + j, prv)
                twf_p[j].start()

        for j in range(S):
            afwd[j].wait_recv()
            bfwd[j].wait_recv()
        twf_n[0].wait_recv()
        twf_n[1].wait_recv()
        xr3[2].wait_recv()
        xr3[3].wait_recv()
        twf_p[0].wait_recv()
        twf_p[1].wait_recv()
        twf_p[2].wait_recv()
        xr5[3].wait_recv()

        for d in (x1 + own_n + own_p + own_t + afwd + bfwd
                  + list(twf_n.values()) + list(twf_p.values())
                  + list(xr3.values()) + list(xr5.values())):
            d.wait_send()

    return pl.pallas_call(
        body,
        out_shape=jax.ShapeDtypeStruct((M, D), jnp.float32),
        in_specs=[
            pl.BlockSpec(memory_space=pltpu.VMEM),
            pl.BlockSpec(memory_space=pltpu.VMEM),
            pl.BlockSpec(memory_space=pltpu.VMEM),
        ],
        out_specs=pl.BlockSpec(memory_space=pltpu.VMEM),
        scratch_shapes=[
            pltpu.VMEM((CH, D), jnp.float32),
            pltpu.SemaphoreType.DMA((10 * S,)),
            pltpu.SemaphoreType.DMA((10 * S,)),
        ],
        compiler_params=pltpu.CompilerParams(collective_id=0),
    )(partial, resid, gamma2d)
